# Initial kernel scaffold; baseline (speedup 1.0000x reference)
import jax
import jax.numpy as jnp
from jax.experimental import pallas as pl


def kernel(center, positive, negative, in_emb, out_emb):
    raise NotImplementedError("write your pallas kernel here")


# speedup vs baseline: 4.0174x; 4.0174x over previous
import functools

import jax
import jax.numpy as jnp
from jax import lax
from jax.experimental import pallas as pl
from jax.experimental.pallas import tpu as pltpu
from jax.experimental.pallas import tpu_sc as plsc

B = 4096
V = 100000
E = 128
W = 50
NEG = 250
S = 304
NC = 2
NS = 16
NW = NC * NS
BPW = B // NW
SUB = 64
NSUB = BPW // SUB
NG = S // 16


def _sc_dots_body(center_hbm, idx_hbm, in_emb_hbm, out_emb_hbm, dots_hbm,
                  cidx_v, idx0_v, idx1_v, idx2_v, crows_v, rows_v, part_v,
                  dots_v, sem, isem):
    cid = lax.axis_index("c")
    sid = lax.axis_index("s")
    wid = sid * NC + cid
    base = wid * BPW
    lanes = lax.iota(jnp.int32, 16)

    def sub_body(subi, carry):
        sbase = base + subi * SUB
        pltpu.sync_copy(center_hbm.at[pl.ds(sbase, SUB)], cidx_v)
        pltpu.async_copy(in_emb_hbm.at[cidx_v], crows_v, sem).wait()

        def b_body(bl, carry2):
            o = (sbase + bl) * S
            i0 = pltpu.async_copy(idx_hbm.at[pl.ds(o, 128)], idx0_v, isem)
            i1 = pltpu.async_copy(idx_hbm.at[pl.ds(o + 128, 128)], idx1_v,
                                  isem)
            i2 = pltpu.async_copy(idx_hbm.at[pl.ds(o + 256, S - 256)],
                                  idx2_v, isem)
            i0.wait()
            i1.wait()
            i2.wait()
            cp0 = pltpu.async_copy(
                out_emb_hbm.at[idx0_v], rows_v.at[pl.ds(0, 128)], sem)
            cp1 = pltpu.async_copy(
                out_emb_hbm.at[idx1_v], rows_v.at[pl.ds(128, 128)], sem)
            cp2 = pltpu.async_copy(
                out_emb_hbm.at[idx2_v], rows_v.at[pl.ds(256, S - 256)], sem)
            cp0.wait()
            cp1.wait()
            cp2.wait()

            c_chunks = [crows_v[bl, pl.ds(k * 16, 16)] for k in range(E // 16)]

            def g_body(g, carry3):
                r0 = g * 16
                for l in range(16):
                    r = r0 + l
                    s = rows_v[r, pl.ds(0, 16)] * c_chunks[0]
                    for k in range(1, E // 16):
                        s = s + rows_v[r, pl.ds(k * 16, 16)] * c_chunks[k]
                    part_v[l, :] = s
                acc = plsc.load_gather(
                    part_v, [lanes, jnp.zeros((16,), jnp.int32)])
                for c in range(1, 16):
                    acc = acc + plsc.load_gather(
                        part_v, [lanes, jnp.full((16,), c, jnp.int32)])
                dots_v[pl.ds(bl * S + r0, 16)] = acc
                return carry3

            lax.fori_loop(0, NG, g_body, 0)
            return carry2

        lax.fori_loop(0, SUB, b_body, 0)
        pltpu.sync_copy(dots_v, dots_hbm.at[pl.ds(sbase * S, SUB * S)])
        return carry

    lax.fori_loop(0, NSUB, sub_body, 0)


@functools.partial(jax.jit, static_argnames=("interpret",))
def _sc_dots(center, idx_flat, in_emb, out_emb, interpret=False):
    mesh = plsc.VectorSubcoreMesh(core_axis_name="c", subcore_axis_name="s",
                                  num_cores=NC, num_subcores=NS)
    return pl.kernel(
        _sc_dots_body,
        out_type=jax.ShapeDtypeStruct((B * S,), jnp.float32),
        mesh=mesh,
        scratch_types=[
            pltpu.VMEM((SUB,), jnp.int32),
            pltpu.VMEM((128,), jnp.int32),
            pltpu.VMEM((128,), jnp.int32),
            pltpu.VMEM((S - 256,), jnp.int32),
            pltpu.VMEM((SUB, E), jnp.float32),
            pltpu.VMEM((S, E), jnp.float32),
            pltpu.VMEM((16, 16), jnp.float32),
            pltpu.VMEM((SUB * S,), jnp.float32),
            pltpu.SemaphoreType.DMA,
            pltpu.SemaphoreType.DMA,
        ],
        compiler_params=pltpu.CompilerParams(needs_layout_passes=False),
        interpret=interpret,
    )(center, idx_flat, in_emb, out_emb)


BI = 256


def _tc_loss_body(dots_ref, pw_ref, out_ref, pos_s, neg_s):
    i = pl.program_id(0)

    @pl.when(i == 0)
    def _():
        d = dots_ref[...]
        pos = jax.nn.log_sigmoid(d[:, :W]) * pw_ref[...]
        neg = jax.nn.log_sigmoid(-d[:, W:W + NEG])
        pos_s[...] = jnp.sum(pos, axis=1)[None, :]
        neg_s[...] = jnp.sum(neg, axis=1)[None, :]

    nb = neg_s[0, pl.ds(i * BI, BI)]
    out_ref[...] = -(pos_s[...] + nb[:, None])


def _tc_loss(dots2d, pw, interpret=False):
    return pl.pallas_call(
        _tc_loss_body,
        grid=(B // BI,),
        in_specs=[
            pl.BlockSpec((B, S), lambda i: (0, 0)),
            pl.BlockSpec((B, W), lambda i: (0, 0)),
        ],
        out_specs=pl.BlockSpec((BI, B), lambda i: (i, 0)),
        out_shape=jax.ShapeDtypeStruct((B, B), jnp.float32),
        scratch_shapes=[
            pltpu.VMEM((1, B), jnp.float32),
            pltpu.VMEM((1, B), jnp.float32),
        ],
        interpret=interpret,
    )(dots2d, pw)


def kernel(center, positive, negative, in_emb, out_emb, *, _interpret=False):
    center = center.astype(jnp.int32)
    pos_idx = positive[:, :, 0].astype(jnp.int32)
    pw = positive[:, :, 1].astype(jnp.float32)
    neg_idx = negative.astype(jnp.int32)
    idx_flat = jnp.concatenate(
        [pos_idx, neg_idx, jnp.zeros((B, S - W - NEG), jnp.int32)],
        axis=1).reshape(-1)
    dots = _sc_dots(center, idx_flat, in_emb, out_emb, interpret=_interpret)
    return _tc_loss(dots.reshape(B, S), pw, interpret=_interpret)


# speedup vs baseline: 4.9345x; 1.2283x over previous
import functools

import jax
import jax.numpy as jnp
from jax import lax
from jax.experimental import pallas as pl
from jax.experimental.pallas import tpu as pltpu
from jax.experimental.pallas import tpu_sc as plsc

B = 4096
V = 100000
E = 128
W = 50
NEG = 250
S = 304
NC = 2
NS = 16
NW = NC * NS
BPW = B // NW
SUB = 32
NSUB = BPW // SUB
NG = S // 16


def _sc_dots_body(center_hbm, idx_hbm, in_emb_hbm, out_emb_hbm, dots_hbm,
                  cidx_v, idx_v, crows_v, rows_a, rows_b, part_v, dots_v,
                  sem_a, sem_b, csem):
    cid = lax.axis_index("c")
    sid = lax.axis_index("s")
    wid = sid * NC + cid
    base = wid * BPW
    lanes = lax.iota(jnp.int32, 16)

    def fire(bl, rows_buf, sem):
        o = bl * S
        pltpu.async_copy(out_emb_hbm.at[idx_v.at[pl.ds(o, 128)]],
                         rows_buf.at[pl.ds(0, 128)], sem)
        pltpu.async_copy(out_emb_hbm.at[idx_v.at[pl.ds(o + 128, 128)]],
                         rows_buf.at[pl.ds(128, 128)], sem)
        pltpu.async_copy(out_emb_hbm.at[idx_v.at[pl.ds(o + 256, S - 256)]],
                         rows_buf.at[pl.ds(256, S - 256)], sem)

    def drain(rows_buf, sem):
        pltpu.make_async_copy(out_emb_hbm.at[pl.ds(0, S)], rows_buf,
                              sem).wait()

    def compute(bl, rows_buf):
        c_chunks = [crows_v[bl, pl.ds(k * 16, 16)] for k in range(E // 16)]

        def g_body(g, carry3):
            r0 = g * 16
            for l in range(16):
                r = r0 + l
                s = rows_buf[r, pl.ds(0, 16)] * c_chunks[0]
                for k in range(1, E // 16):
                    s = s + rows_buf[r, pl.ds(k * 16, 16)] * c_chunks[k]
                part_v[l, :] = s
            acc = plsc.load_gather(
                part_v, [lanes, jnp.zeros((16,), jnp.int32)])
            for c in range(1, 16):
                acc = acc + plsc.load_gather(
                    part_v, [lanes, jnp.full((16,), c, jnp.int32)])
            dots_v[pl.ds(bl * S + r0, 16)] = acc
            return carry3

        lax.fori_loop(0, NG, g_body, 0)

    def sub_body(subi, carry):
        sbase = base + subi * SUB
        pltpu.sync_copy(center_hbm.at[pl.ds(sbase, SUB)], cidx_v)
        pltpu.sync_copy(idx_hbm.at[pl.ds(sbase * S, SUB * S)], idx_v)
        pltpu.async_copy(in_emb_hbm.at[cidx_v], crows_v, csem).wait()

        fire(0, rows_a, sem_a)
        fire(1, rows_b, sem_b)

        def pair_body(bp, carry2):
            b0 = 2 * bp
            b1 = b0 + 1
            drain(rows_a, sem_a)
            compute(b0, rows_a)

            @pl.when(b0 + 2 < SUB)
            def _():
                fire(b0 + 2, rows_a, sem_a)

            drain(rows_b, sem_b)
            compute(b1, rows_b)

            @pl.when(b1 + 2 < SUB)
            def _():
                fire(b1 + 2, rows_b, sem_b)

            return carry2

        lax.fori_loop(0, SUB // 2, pair_body, 0)
        pltpu.sync_copy(dots_v, dots_hbm.at[pl.ds(sbase * S, SUB * S)])
        return carry

    lax.fori_loop(0, NSUB, sub_body, 0)


@functools.partial(jax.jit, static_argnames=("interpret",))
def _sc_dots(center, idx_flat, in_emb, out_emb, interpret=False):
    mesh = plsc.VectorSubcoreMesh(core_axis_name="c", subcore_axis_name="s",
                                  num_cores=NC, num_subcores=NS)
    return pl.kernel(
        _sc_dots_body,
        out_type=jax.ShapeDtypeStruct((B * S,), jnp.float32),
        mesh=mesh,
        scratch_types=[
            pltpu.VMEM((SUB,), jnp.int32),
            pltpu.VMEM((SUB * S,), jnp.int32),
            pltpu.VMEM((SUB, E), jnp.float32),
            pltpu.VMEM((S, E), jnp.float32),
            pltpu.VMEM((S, E), jnp.float32),
            pltpu.VMEM((16, 16), jnp.float32),
            pltpu.VMEM((SUB * S,), jnp.float32),
            pltpu.SemaphoreType.DMA,
            pltpu.SemaphoreType.DMA,
            pltpu.SemaphoreType.DMA,
        ],
        compiler_params=pltpu.CompilerParams(needs_layout_passes=False),
        interpret=interpret,
    )(center, idx_flat, in_emb, out_emb)


BI = 256


def _tc_loss_body(dots_ref, pw_ref, out_ref, pos_s, neg_s):
    i = pl.program_id(0)

    @pl.when(i == 0)
    def _():
        d = dots_ref[...]
        pos = jax.nn.log_sigmoid(d[:, :W]) * pw_ref[...]
        neg = jax.nn.log_sigmoid(-d[:, W:W + NEG])
        pos_s[...] = jnp.sum(pos, axis=1)[None, :]
        neg_s[...] = jnp.sum(neg, axis=1)[None, :]

    nb = neg_s[0, pl.ds(i * BI, BI)]
    out_ref[...] = -(pos_s[...] + nb[:, None])


def _tc_loss(dots2d, pw, interpret=False):
    return pl.pallas_call(
        _tc_loss_body,
        grid=(B // BI,),
        in_specs=[
            pl.BlockSpec((B, S), lambda i: (0, 0)),
            pl.BlockSpec((B, W), lambda i: (0, 0)),
        ],
        out_specs=pl.BlockSpec((BI, B), lambda i: (i, 0)),
        out_shape=jax.ShapeDtypeStruct((B, B), jnp.float32),
        scratch_shapes=[
            pltpu.VMEM((1, B), jnp.float32),
            pltpu.VMEM((1, B), jnp.float32),
        ],
        interpret=interpret,
    )(dots2d, pw)


def kernel(center, positive, negative, in_emb, out_emb, *, _interpret=False):
    center = center.astype(jnp.int32)
    pos_idx = positive[:, :, 0].astype(jnp.int32)
    pw = positive[:, :, 1].astype(jnp.float32)
    neg_idx = negative.astype(jnp.int32)
    idx_flat = jnp.concatenate(
        [pos_idx, neg_idx, jnp.zeros((B, S - W - NEG), jnp.int32)],
        axis=1).reshape(-1)
    dots = _sc_dots(center, idx_flat, in_emb, out_emb, interpret=_interpret)
    return _tc_loss(dots.reshape(B, S), pw, interpret=_interpret)
